# baseline (speedup 1.0000x reference)
import jax
import jax.numpy as jnp
from jax import lax
from jax.experimental import pallas as pl
from jax.experimental.pallas import tpu as pltpu

V0, V1, V2 = 3584, 1536, 512
E = 128
B = 1024
LD, LP = 32, 16

V0P, V1P, V2P = 3584, 1536, 512
W = 1024
R = 6040
OFF_EMB0 = 0
OFF_EMB1 = 3584
OFF_WH = 5120
OFF_BH = 5376
OFF_W1 = 5384
OFF_B1 = 5512
OFF_W2 = 5520
OFF_B2 = 6032
MLP_ROWS = R - OFF_WH
M_WH = OFF_WH - OFF_WH
M_W1 = OFF_W1 - OFF_WH
M_W2 = OFF_W2 - OFF_WH

TB = 512
CONTRACT0 = (((0,), (0,)), ((), ()))


def _body(didx_ref, pidx_ref, p_hbm, out_ref,
          emb0_s, emb1_s, mlp_s, dcnt_ref, pcnt_ref, sem):
    f32 = jnp.float32
    bf16 = jnp.bfloat16
    i16 = jnp.int16

    cp0 = pltpu.make_async_copy(
        p_hbm.at[pl.ds(OFF_EMB0, V0P), pl.ds(0, E)], emb0_s, sem.at[0])
    cp1 = pltpu.make_async_copy(
        p_hbm.at[pl.ds(OFF_EMB1, V1P), pl.ds(0, E)], emb1_s, sem.at[1])
    cp2 = pltpu.make_async_copy(
        p_hbm.at[pl.ds(OFF_WH, MLP_ROWS), pl.ds(0, W)], mlp_s, sem.at[2])
    cp0.start()
    cp1.start()
    cp2.start()

    def bag(idx_ref, cnt_ref, ntaps, vocab_p):
        row = lax.broadcasted_iota(i16, (vocab_p, TB), 0)
        cnt_ref[...] = jnp.zeros((vocab_p, TB), i16)

        def tap8(i, c):
            v8 = idx_ref[pl.ds(i * 8, 8), :].astype(i16)
            m = (row == v8[0:1, :]).astype(i16)
            for j in range(1, 8):
                m = m + (row == v8[j:j + 1, :]).astype(i16)
            cnt_ref[...] = cnt_ref[...] + m
            return c

        lax.fori_loop(0, ntaps // 8, tap8, 0)

    bag(didx_ref, dcnt_ref, LD, V0P)
    cp0.wait()
    dsumT = lax.dot_general(emb0_s[...].astype(bf16),
                            dcnt_ref[...].astype(bf16),
                            CONTRACT0, preferred_element_type=f32)
    bag(pidx_ref, pcnt_ref, LP, V1P)
    cp1.wait()
    psumT = lax.dot_general(emb1_s[...].astype(bf16),
                            pcnt_ref[...].astype(bf16),
                            CONTRACT0, preferred_element_type=f32)

    ones8 = jnp.ones((8, TB), f32)
    cp2.wait()
    hrT = jnp.concatenate([dsumT, psumT, ones8], axis=0).astype(bf16)
    wh = mlp_s[M_WH:M_WH + 2 * E + 8, :E].astype(bf16)
    repT = lax.dot_general(wh, hrT, CONTRACT0,
                           preferred_element_type=f32)

    rep_aug = jnp.concatenate([repT, ones8], axis=0).astype(bf16)
    w1 = mlp_s[M_W1:M_W1 + E + 8, :8 * E].astype(bf16)
    hT = jnp.maximum(
        lax.dot_general(w1, rep_aug, CONTRACT0,
                        preferred_element_type=f32), 0.0)

    ha = jnp.concatenate([hT[:4 * E, :], ones8], axis=0).astype(bf16)
    hd = jnp.concatenate([hT[4 * E:, :], ones8], axis=0).astype(bf16)
    w2a = mlp_s[M_W2:M_W2 + 4 * E + 8, 0:V2P].astype(bf16)
    w2d = mlp_s[M_W2:M_W2 + 4 * E + 8, V2P:2 * V2P].astype(bf16)
    o_addT = lax.dot_general(w2a, ha, CONTRACT0,
                             preferred_element_type=f32)
    o_delT = lax.dot_general(w2d, hd, CONTRACT0,
                             preferred_element_type=f32)

    out_ref[0:V2P, :] = o_delT
    out_ref[V2P:2 * V2P, :] = o_addT


_call = pl.pallas_call(
    _body,
    grid=(B // TB,),
    in_specs=[
        pl.BlockSpec((LD, TB), lambda g: (0, g)),
        pl.BlockSpec((LP, TB), lambda g: (0, g)),
        pl.BlockSpec(memory_space=pl.ANY),
    ],
    out_specs=pl.BlockSpec((2 * V2P, TB), lambda g: (0, g)),
    out_shape=jax.ShapeDtypeStruct((2 * V2P, B), jnp.float32),
    scratch_shapes=[pltpu.VMEM((V0P, E), jnp.float32),
                    pltpu.VMEM((V1P, E), jnp.float32),
                    pltpu.VMEM((MLP_ROWS, W), jnp.float32),
                    pltpu.VMEM((V0P, TB), jnp.int16),
                    pltpu.VMEM((V1P, TB), jnp.int16),
                    pltpu.SemaphoreType.DMA((3,))],
    compiler_params=pltpu.CompilerParams(
        dimension_semantics=("parallel",)),
)


@jax.jit
def _forward(packed, diag_idx, prod_idx):
    diagT = jnp.asarray(diag_idx, jnp.int32).T
    prodT = jnp.asarray(prod_idx, jnp.int32).T
    raw = _call(diagT, prodT, packed)
    o_del = raw[0:V2P]
    o_add = raw[V2P:2 * V2P]
    out = jnp.stack([o_del, o_add, o_del], axis=0)
    return jnp.transpose(out, (2, 1, 0))


def kernel(packed, diag_idx, prod_idx):
    return _forward(packed, diag_idx, prod_idx)


# speedup vs baseline: 1.0731x; 1.0731x over previous
import jax
import jax.numpy as jnp
from jax import lax
from jax.experimental import pallas as pl
from jax.experimental.pallas import tpu as pltpu

V0, V1, V2 = 3584, 1536, 512
E = 128
B = 1024
LD, LP = 32, 16

V0P, V1P, V2P = 3584, 1536, 512
W = 1024
R = 6040
OFF_EMB0 = 0
OFF_EMB1 = 3584
OFF_WH = 5120
OFF_BH = 5376
OFF_W1 = 5384
OFF_B1 = 5512
OFF_W2 = 5520
OFF_B2 = 6032
MLP_ROWS = R - OFF_WH
M_WH = OFF_WH - OFF_WH
M_W1 = OFF_W1 - OFF_WH
M_W2 = OFF_W2 - OFF_WH

TB = 512
CONTRACT0 = (((0,), (0,)), ((), ()))


def _body(didx_ref, pidx_ref, p_hbm, out_ref,
          emb0_s, emb1_s, mlp_s, dcnt_ref, pcnt_ref, sem):
    f32 = jnp.float32
    bf16 = jnp.bfloat16
    i16 = jnp.int16

    cp0 = pltpu.make_async_copy(
        p_hbm.at[pl.ds(OFF_EMB0, V0P), pl.ds(0, E)], emb0_s, sem.at[0])
    cp1 = pltpu.make_async_copy(
        p_hbm.at[pl.ds(OFF_EMB1, V1P), pl.ds(0, E)], emb1_s, sem.at[1])
    cp2 = pltpu.make_async_copy(
        p_hbm.at[pl.ds(OFF_WH, MLP_ROWS), pl.ds(0, W)], mlp_s, sem.at[2])
    cp0.start()
    cp1.start()
    cp2.start()

    def bag(idx_ref, cnt_ref, ntaps, vocab_p):
        row = lax.broadcasted_iota(i16, (vocab_p, TB), 0)
        cnt_ref[...] = jnp.zeros((vocab_p, TB), i16)

        def tap16(i, c):
            v16 = idx_ref[pl.ds(i * 16, 16), :].astype(i16)
            m = (row == v16[0:1, :]).astype(i16)
            for j in range(1, 16):
                m = m + (row == v16[j:j + 1, :]).astype(i16)
            cnt_ref[...] = cnt_ref[...] + m
            return c

        lax.fori_loop(0, ntaps // 16, tap16, 0)

    bag(didx_ref, dcnt_ref, LD, V0P)
    cp0.wait()
    dsumT = lax.dot_general(emb0_s[...].astype(bf16),
                            dcnt_ref[...].astype(bf16),
                            CONTRACT0, preferred_element_type=f32)
    bag(pidx_ref, pcnt_ref, LP, V1P)
    cp1.wait()
    psumT = lax.dot_general(emb1_s[...].astype(bf16),
                            pcnt_ref[...].astype(bf16),
                            CONTRACT0, preferred_element_type=f32)

    ones8 = jnp.ones((8, TB), f32)
    cp2.wait()
    hrT = jnp.concatenate([dsumT, psumT, ones8], axis=0).astype(bf16)
    wh = mlp_s[M_WH:M_WH + 2 * E + 8, :E].astype(bf16)
    repT = lax.dot_general(wh, hrT, CONTRACT0,
                           preferred_element_type=f32)

    rep_aug = jnp.concatenate([repT, ones8], axis=0).astype(bf16)
    w1 = mlp_s[M_W1:M_W1 + E + 8, :8 * E].astype(bf16)
    hT = jnp.maximum(
        lax.dot_general(w1, rep_aug, CONTRACT0,
                        preferred_element_type=f32), 0.0)

    ha = jnp.concatenate([hT[:4 * E, :], ones8], axis=0).astype(bf16)
    hd = jnp.concatenate([hT[4 * E:, :], ones8], axis=0).astype(bf16)
    w2a = mlp_s[M_W2:M_W2 + 4 * E + 8, 0:V2P].astype(bf16)
    w2d = mlp_s[M_W2:M_W2 + 4 * E + 8, V2P:2 * V2P].astype(bf16)
    o_addT = lax.dot_general(w2a, ha, CONTRACT0,
                             preferred_element_type=f32)
    o_delT = lax.dot_general(w2d, hd, CONTRACT0,
                             preferred_element_type=f32)

    out_ref[0:V2P, :] = o_delT
    out_ref[V2P:2 * V2P, :] = o_addT
    out_ref[2 * V2P:3 * V2P, :] = o_delT


_call = pl.pallas_call(
    _body,
    grid=(B // TB,),
    in_specs=[
        pl.BlockSpec((LD, TB), lambda g: (0, g)),
        pl.BlockSpec((LP, TB), lambda g: (0, g)),
        pl.BlockSpec(memory_space=pl.ANY),
    ],
    out_specs=pl.BlockSpec((3 * V2P, TB), lambda g: (0, g)),
    out_shape=jax.ShapeDtypeStruct((3 * V2P, B), jnp.float32),
    scratch_shapes=[pltpu.VMEM((V0P, E), jnp.float32),
                    pltpu.VMEM((V1P, E), jnp.float32),
                    pltpu.VMEM((MLP_ROWS, W), jnp.float32),
                    pltpu.VMEM((V0P, TB), jnp.int16),
                    pltpu.VMEM((V1P, TB), jnp.int16),
                    pltpu.SemaphoreType.DMA((3,))],
    compiler_params=pltpu.CompilerParams(
        dimension_semantics=("parallel",)),
)


@jax.jit
def _forward(packed, diag_idx, prod_idx):
    diagT = jnp.asarray(diag_idx, jnp.int32).T
    prodT = jnp.asarray(prod_idx, jnp.int32).T
    raw = _call(diagT, prodT, packed)
    out = raw.reshape(3, V2P, B)[:, :V2, :]
    return jnp.transpose(out, (2, 1, 0))


def kernel(packed, diag_idx, prod_idx):
    return _forward(packed, diag_idx, prod_idx)


# speedup vs baseline: 1.0959x; 1.0212x over previous
import jax
import jax.numpy as jnp
from jax import lax
from jax.experimental import pallas as pl
from jax.experimental.pallas import tpu as pltpu

V0, V1, V2 = 3584, 1536, 512
E = 128
B = 1024
LD, LP = 32, 16

V0P, V1P, V2P = 3584, 1536, 512
W = 1024
R = 6040
OFF_EMB0 = 0
OFF_EMB1 = 3584
OFF_WH = 5120
OFF_BH = 5376
OFF_W1 = 5384
OFF_B1 = 5512
OFF_W2 = 5520
OFF_B2 = 6032
MLP_ROWS = R - OFF_WH
M_WH = OFF_WH - OFF_WH
M_W1 = OFF_W1 - OFF_WH
M_W2 = OFF_W2 - OFF_WH

TB = 512
CONTRACT0 = (((0,), (0,)), ((), ()))


def _body(didx_ref, pidx_ref, p_hbm, out_ref,
          emb0_s, emb1_s, mlp_s, dcnt_ref, pcnt_ref, sem):
    f32 = jnp.float32
    bf16 = jnp.bfloat16
    i16 = jnp.int16

    cp0 = pltpu.make_async_copy(
        p_hbm.at[pl.ds(OFF_EMB0, V0P), pl.ds(0, E)], emb0_s, sem.at[0])
    cp1 = pltpu.make_async_copy(
        p_hbm.at[pl.ds(OFF_EMB1, V1P), pl.ds(0, E)], emb1_s, sem.at[1])
    cp2 = pltpu.make_async_copy(
        p_hbm.at[pl.ds(OFF_WH, MLP_ROWS), pl.ds(0, W)], mlp_s, sem.at[2])
    cp0.start()
    cp1.start()
    cp2.start()

    def bag(idx_ref, cnt_ref, ntaps, vocab_p):
        row = lax.broadcasted_iota(i16, (vocab_p, TB), 0)
        cnt_ref[...] = jnp.zeros((vocab_p, TB), i16)

        def tap8(i, c):
            v8 = idx_ref[pl.ds(i * 8, 8), :].astype(i16)
            m = (row == v8[0:1, :]).astype(i16)
            for j in range(1, 8):
                m = m + (row == v8[j:j + 1, :]).astype(i16)
            cnt_ref[...] = cnt_ref[...] + m
            return c

        lax.fori_loop(0, ntaps // 8, tap8, 0)

    bag(didx_ref, dcnt_ref, LD, V0P)
    cp0.wait()
    dsumT = lax.dot_general(emb0_s[...].astype(bf16),
                            dcnt_ref[...].astype(bf16),
                            CONTRACT0, preferred_element_type=f32)
    bag(pidx_ref, pcnt_ref, LP, V1P)
    cp1.wait()
    psumT = lax.dot_general(emb1_s[...].astype(bf16),
                            pcnt_ref[...].astype(bf16),
                            CONTRACT0, preferred_element_type=f32)

    ones8 = jnp.ones((8, TB), f32)
    cp2.wait()
    hrT = jnp.concatenate([dsumT, psumT, ones8], axis=0).astype(bf16)
    wh = mlp_s[M_WH:M_WH + 2 * E + 8, :E].astype(bf16)
    repT = lax.dot_general(wh, hrT, CONTRACT0,
                           preferred_element_type=f32)

    rep_aug = jnp.concatenate([repT, ones8], axis=0).astype(bf16)
    w1 = mlp_s[M_W1:M_W1 + E + 8, :8 * E].astype(bf16)
    hT = jnp.maximum(
        lax.dot_general(w1, rep_aug, CONTRACT0,
                        preferred_element_type=f32), 0.0)

    ha = jnp.concatenate([hT[:4 * E, :], ones8], axis=0).astype(bf16)
    hd = jnp.concatenate([hT[4 * E:, :], ones8], axis=0).astype(bf16)
    w2a = mlp_s[M_W2:M_W2 + 4 * E + 8, 0:V2P].astype(bf16)
    w2d = mlp_s[M_W2:M_W2 + 4 * E + 8, V2P:2 * V2P].astype(bf16)
    o_addT = lax.dot_general(w2a, ha, CONTRACT0,
                             preferred_element_type=f32)
    o_delT = lax.dot_general(w2d, hd, CONTRACT0,
                             preferred_element_type=f32)

    out_ref[0:V2P, :] = o_delT
    out_ref[V2P:2 * V2P, :] = o_addT
    out_ref[2 * V2P:3 * V2P, :] = o_delT


_call = pl.pallas_call(
    _body,
    grid=(B // TB,),
    in_specs=[
        pl.BlockSpec((LD, TB), lambda g: (0, g)),
        pl.BlockSpec((LP, TB), lambda g: (0, g)),
        pl.BlockSpec(memory_space=pl.ANY),
    ],
    out_specs=pl.BlockSpec((3 * V2P, TB), lambda g: (0, g)),
    out_shape=jax.ShapeDtypeStruct((3 * V2P, B), jnp.float32),
    scratch_shapes=[pltpu.VMEM((V0P, E), jnp.float32),
                    pltpu.VMEM((V1P, E), jnp.float32),
                    pltpu.VMEM((MLP_ROWS, W), jnp.float32),
                    pltpu.VMEM((V0P, TB), jnp.int16),
                    pltpu.VMEM((V1P, TB), jnp.int16),
                    pltpu.SemaphoreType.DMA((3,))],
    compiler_params=pltpu.CompilerParams(
        dimension_semantics=("parallel",)),
)


@jax.jit
def _forward(packed, diag_idx, prod_idx):
    diagT = jnp.asarray(diag_idx, jnp.int32).T
    prodT = jnp.asarray(prod_idx, jnp.int32).T
    raw = _call(diagT, prodT, packed)
    out = raw.reshape(3, V2P, B)[:, :V2, :]
    return jnp.transpose(out, (2, 1, 0))


def kernel(packed, diag_idx, prod_idx):
    return _forward(packed, diag_idx, prod_idx)
